# baseline (speedup 1.0000x reference)
import functools

import jax
import jax.numpy as jnp
from jax import lax
from jax.experimental import pallas as pl
from jax.experimental.pallas import tpu as pltpu
from jax.experimental.pallas import tpu_sc as plsc

S, D, FF, E, TOPK = 2048, 1024, 2048, 8, 2
EP = 128
TILE = 128
NP = S * TOPK
NT = (NP + E * (TILE - 1) + TILE - 1) // TILE
NR = NT * TILE
_NEG = -1e30

NC, NS = 2, 16
NW = NC * NS
TPW = S // NW
CH = 32


def _gate_body(x_ref, tp_ref, wgi_ref, wgt_ref, a_ref,
               post_ref, pos_ref, wt_ref, te_ref, laux_ref):
    a = a_ref[0, 0]
    gi = jnp.dot(x_ref[...], wgi_ref[...], preferred_element_type=jnp.float32)
    gt = jnp.dot(tp_ref[...], wgt_ref[...], preferred_element_type=jnp.float32)
    g = (1.0 - a) * gi + a * gt
    lane = lax.broadcasted_iota(jnp.int32, (S, E), 1)
    m1 = jnp.max(g, axis=1, keepdims=True)
    i1 = jnp.min(jnp.where(g == m1, lane, E), axis=1, keepdims=True)
    g2 = jnp.where(lane == i1, _NEG, g)
    m2 = jnp.max(g2, axis=1, keepdims=True)
    i2 = jnp.min(jnp.where(g2 == m2, lane, E), axis=1, keepdims=True)
    p = jnp.exp(g - m1)
    p = p / jnp.sum(p, axis=1, keepdims=True)
    aw = jnp.mean(p, axis=0, keepdims=True)
    cnt1 = (lane == i1).astype(jnp.float32)
    cnt2 = (lane == i2).astype(jnp.float32)
    ac = jnp.mean(cnt1 + cnt2, axis=0, keepdims=True)
    laux_ref[0, 0] = jnp.sum(aw * ac)
    e21 = jnp.exp(m2 - m1)
    wt_ref[...] = jnp.concatenate([1.0 / (1.0 + e21), e21 / (1.0 + e21)], axis=1)
    tot = (cnt1 + cnt2).astype(jnp.bfloat16)
    iota_r = lax.broadcasted_iota(jnp.int32, (S, S), 0)
    iota_c = lax.broadcasted_iota(jnp.int32, (S, S), 1)
    lower = (iota_c < iota_r).astype(jnp.bfloat16)
    P = jnp.dot(lower, tot, preferred_element_type=jnp.float32)
    counts = P[S - 1:S] + tot[S - 1:S].astype(jnp.float32)
    ptiles_f = jnp.ceil(counts * (1.0 / TILE))
    incl = (lax.broadcasted_iota(jnp.int32, (E, E), 0)
            <= lax.broadcasted_iota(jnp.int32, (E, E), 1)).astype(jnp.bfloat16)
    pe_t = jnp.dot(ptiles_f.astype(jnp.bfloat16), incl,
                   preferred_element_type=jnp.float32)
    pad_end = pe_t * TILE
    pad_off = pad_end - ptiles_f * TILE
    pos1 = jnp.sum(cnt1 * (pad_off + P), axis=1, keepdims=True)
    pos2 = jnp.sum(cnt2 * (pad_off + P), axis=1, keepdims=True)
    pos_ref[...] = jnp.concatenate([pos1, pos2], axis=1).astype(jnp.int32)
    a1 = jnp.floor(pos1 * (1.0 / TILE))
    a2f = jnp.floor(pos2 * (1.0 / TILE))
    dig = jnp.concatenate([a1, pos1 - a1 * TILE, a2f, pos2 - a2f * TILE],
                          axis=1).astype(jnp.bfloat16)
    ident = (iota_c == iota_r).astype(jnp.bfloat16)
    digT = lax.dot_general(dig, ident, (((0,), (0,)), ((), ())),
                           preferred_element_type=jnp.float32)
    post_ref[...] = jnp.concatenate(
        [digT[0:1] * TILE + digT[1:2], digT[2:3] * TILE + digT[3:4]],
        axis=0).astype(jnp.int32)
    tvec = lax.broadcasted_iota(jnp.int32, (EP, E), 0).astype(jnp.float32) * TILE
    amat = (tvec >= pad_end).astype(jnp.int32)
    te = jnp.minimum(jnp.sum(amat, axis=1, keepdims=True), E - 1)
    nact = jnp.broadcast_to(pe_t[:, E - 1:E], (EP, 1)).astype(jnp.int32)
    rowi = lax.broadcasted_iota(jnp.int32, (EP, 1), 0)
    te_ref[...] = jnp.where(rowi == 64, nact, te)


def _dispatch_body(x_hbm, post_hbm, xg_hbm, idx1_v, idx2_v, rows_v, sem1, sem2):
    wid = lax.axis_index("s") * NC + lax.axis_index("c")
    base0 = wid * TPW
    for chunk in range(TPW // CH):
        base = base0 + chunk * CH
        pltpu.sync_copy(post_hbm.at[0, pl.ds(base, CH)], idx1_v)
        pltpu.sync_copy(post_hbm.at[1, pl.ds(base, CH)], idx2_v)
        pltpu.sync_copy(x_hbm.at[pl.ds(base, CH)], rows_v)
        cp1 = pltpu.async_copy(rows_v, xg_hbm.at[idx1_v], sem1)
        cp2 = pltpu.async_copy(rows_v, xg_hbm.at[idx2_v], sem2)
        cp1.wait()
        cp2.wait()


def _ffn_body(te_ref, xg_ref, w1_ref, w2_ref, pos_ref, wt_ref, out_ref):
    t = pl.program_id(0)

    @pl.when(t == 0)
    def _():
        out_ref[...] = jnp.zeros_like(out_ref)

    @pl.when(t < te_ref[64, 0])
    def _():
        base = t * TILE
        coliota = lax.broadcasted_iota(jnp.int32, (S, TILE), 1) + base
        c1 = coliota == pos_ref[:, 0:1]
        c2 = coliota == pos_ref[:, 1:2]
        ohw = (jnp.where(c1, wt_ref[:, 0:1], 0.0)
               + jnp.where(c2, wt_ref[:, 1:2], 0.0)).astype(jnp.bfloat16)
        h = jnp.dot(xg_ref[...], w1_ref[0], preferred_element_type=jnp.float32)
        h = h * lax.logistic(h)
        y = jnp.dot(h, w2_ref[0], preferred_element_type=jnp.float32)
        out_ref[...] += jnp.dot(ohw, y, preferred_element_type=jnp.float32)


def kernel(inputs, task_param, Wg_in, Wg_task, W1, W2, alpha):
    x = inputs.reshape(S, D)
    tp = task_param.reshape(S, D)
    a2 = jnp.reshape(alpha.astype(jnp.float32), (1, 1))

    post, pos, wt, te, laux = pl.pallas_call(
        _gate_body,
        in_specs=[
            pl.BlockSpec((S, D), lambda: (0, 0)),
            pl.BlockSpec((S, D), lambda: (0, 0)),
            pl.BlockSpec((D, E), lambda: (0, 0)),
            pl.BlockSpec((D, E), lambda: (0, 0)),
            pl.BlockSpec(memory_space=pltpu.SMEM),
        ],
        out_specs=[
            pl.BlockSpec((TOPK, S), lambda: (0, 0)),
            pl.BlockSpec((S, TOPK), lambda: (0, 0)),
            pl.BlockSpec((S, TOPK), lambda: (0, 0)),
            pl.BlockSpec((EP, 1), lambda: (0, 0)),
            pl.BlockSpec(memory_space=pltpu.SMEM),
        ],
        out_shape=[
            jax.ShapeDtypeStruct((TOPK, S), jnp.int32),
            jax.ShapeDtypeStruct((S, TOPK), jnp.int32),
            jax.ShapeDtypeStruct((S, TOPK), jnp.float32),
            jax.ShapeDtypeStruct((EP, 1), jnp.int32),
            jax.ShapeDtypeStruct((1, 1), jnp.float32),
        ],
    )(x, tp, Wg_in, Wg_task, a2)

    mesh = plsc.VectorSubcoreMesh(core_axis_name="c", subcore_axis_name="s")
    dispatch = functools.partial(
        pl.kernel, mesh=mesh,
        out_type=jax.ShapeDtypeStruct((NR, D), jnp.float32),
        scratch_types=[
            pltpu.VMEM((CH,), jnp.int32),
            pltpu.VMEM((CH,), jnp.int32),
            pltpu.VMEM((CH, D), jnp.float32),
            pltpu.SemaphoreType.DMA,
            pltpu.SemaphoreType.DMA,
        ],
    )(_dispatch_body)
    xg = dispatch(x, post)

    out = pl.pallas_call(
        _ffn_body,
        grid_spec=pltpu.PrefetchScalarGridSpec(
            num_scalar_prefetch=1,
            grid=(NT,),
            in_specs=[
                pl.BlockSpec((TILE, D), lambda t, te: (t, 0)),
                pl.BlockSpec((1, D, FF), lambda t, te: (te[t, 0], 0, 0)),
                pl.BlockSpec((1, FF, D), lambda t, te: (te[t, 0], 0, 0)),
                pl.BlockSpec((S, TOPK), lambda t, te: (0, 0)),
                pl.BlockSpec((S, TOPK), lambda t, te: (0, 0)),
            ],
            out_specs=pl.BlockSpec((S, D), lambda t, te: (0, 0)),
        ),
        out_shape=jax.ShapeDtypeStruct((S, D), jnp.float32),
    )(te, xg, W1, W2, pos, wt)


    return out.reshape(1, S, D), laux.reshape(())


# speedup vs baseline: 1.1021x; 1.1021x over previous
import functools

import jax
import jax.numpy as jnp
from jax import lax
from jax.experimental import pallas as pl
from jax.experimental.pallas import tpu as pltpu
from jax.experimental.pallas import tpu_sc as plsc

S, D, FF, E, TOPK = 2048, 1024, 2048, 8, 2
EP = 128
TILE = 128
NP = S * TOPK
NT = (NP + E * (TILE - 1) + TILE - 1) // TILE
NR = NT * TILE
_NEG = -1e30

NC, NS = 2, 16
NW = NC * NS
TPW = S // NW
CH = 32
CHD = 64


def _gate_body(x_ref, tp_ref, wgi_ref, wgt_ref, a_ref,
               post_ref, w1_ref, w2_ref, te_ref, laux_ref):
    a = a_ref[0, 0]
    gi = jnp.dot(x_ref[...], wgi_ref[...], preferred_element_type=jnp.float32)
    gt = jnp.dot(tp_ref[...], wgt_ref[...], preferred_element_type=jnp.float32)
    g = (1.0 - a) * gi + a * gt
    lane = lax.broadcasted_iota(jnp.int32, (S, E), 1)
    m1 = jnp.max(g, axis=1, keepdims=True)
    i1 = jnp.min(jnp.where(g == m1, lane, E), axis=1, keepdims=True)
    g2 = jnp.where(lane == i1, _NEG, g)
    m2 = jnp.max(g2, axis=1, keepdims=True)
    i2 = jnp.min(jnp.where(g2 == m2, lane, E), axis=1, keepdims=True)
    p = jnp.exp(g - m1)
    p = p / jnp.sum(p, axis=1, keepdims=True)
    aw = jnp.mean(p, axis=0, keepdims=True)
    cnt1 = (lane == i1).astype(jnp.float32)
    cnt2 = (lane == i2).astype(jnp.float32)
    ac = jnp.mean(cnt1 + cnt2, axis=0, keepdims=True)
    laux_ref[0, 0] = jnp.sum(aw * ac)
    e21 = jnp.exp(m2 - m1)
    w1_ref[...] = jnp.broadcast_to(1.0 / (1.0 + e21), (S, 16))
    w2_ref[...] = jnp.broadcast_to(e21 / (1.0 + e21), (S, 16))
    tot = (cnt1 + cnt2).astype(jnp.bfloat16)
    iota_r = lax.broadcasted_iota(jnp.int32, (S, S), 0)
    iota_c = lax.broadcasted_iota(jnp.int32, (S, S), 1)
    lower = (iota_c < iota_r).astype(jnp.bfloat16)
    P = jnp.dot(lower, tot, preferred_element_type=jnp.float32)
    counts = P[S - 1:S] + tot[S - 1:S].astype(jnp.float32)
    ptiles_f = jnp.ceil(counts * (1.0 / TILE))
    incl = (lax.broadcasted_iota(jnp.int32, (E, E), 0)
            <= lax.broadcasted_iota(jnp.int32, (E, E), 1)).astype(jnp.bfloat16)
    pe_t = jnp.dot(ptiles_f.astype(jnp.bfloat16), incl,
                   preferred_element_type=jnp.float32)
    pad_end = pe_t * TILE
    pad_off = pad_end - ptiles_f * TILE
    pos1 = jnp.sum(cnt1 * (pad_off + P), axis=1, keepdims=True)
    pos2 = jnp.sum(cnt2 * (pad_off + P), axis=1, keepdims=True)
    a1 = jnp.floor(pos1 * (1.0 / TILE))
    a2f = jnp.floor(pos2 * (1.0 / TILE))
    dig = jnp.concatenate([a1, pos1 - a1 * TILE, a2f, pos2 - a2f * TILE],
                          axis=1).astype(jnp.bfloat16)
    ident = (iota_c == iota_r).astype(jnp.bfloat16)
    digT = lax.dot_general(dig, ident, (((0,), (0,)), ((), ())),
                           preferred_element_type=jnp.float32)
    post_ref[...] = jnp.concatenate(
        [digT[0:1] * TILE + digT[1:2], digT[2:3] * TILE + digT[3:4]],
        axis=0).astype(jnp.int32)
    tvec = lax.broadcasted_iota(jnp.int32, (EP, E), 0).astype(jnp.float32) * TILE
    amat = (tvec >= pad_end).astype(jnp.int32)
    te = jnp.minimum(jnp.sum(amat, axis=1, keepdims=True), E - 1)
    nact = jnp.broadcast_to(pe_t[:, E - 1:E], (EP, 1)).astype(jnp.int32)
    rowi = lax.broadcasted_iota(jnp.int32, (EP, 1), 0)
    te_ref[...] = jnp.where(rowi == 64, nact, te)


def _dispatch_body(x_hbm, post_hbm, xg_hbm, idx1_v, idx2_v, rows_v, sem1, sem2):
    wid = lax.axis_index("s") * NC + lax.axis_index("c")
    base0 = wid * TPW
    for chunk in range(TPW // CHD):
        base = base0 + chunk * CHD
        pltpu.sync_copy(post_hbm.at[0, pl.ds(base, CHD)], idx1_v)
        pltpu.sync_copy(post_hbm.at[1, pl.ds(base, CHD)], idx2_v)
        pltpu.sync_copy(x_hbm.at[pl.ds(base, CHD)], rows_v)
        cp1 = pltpu.async_copy(rows_v, xg_hbm.at[idx1_v], sem1)
        cp2 = pltpu.async_copy(rows_v, xg_hbm.at[idx2_v], sem2)
        cp1.wait()
        cp2.wait()


def _ffn_body(te_ref, xg_ref, w1_ref, w2_ref, yw_ref):
    t = pl.program_id(0)

    @pl.when(t < te_ref[64, 0])
    def _():
        h = jnp.dot(xg_ref[...], w1_ref[0], preferred_element_type=jnp.float32)
        h = h * lax.logistic(h)
        yw_ref[...] = jnp.dot(h, w2_ref[0], preferred_element_type=jnp.float32)


def _combine_body(yw_hbm, post_hbm, w1_hbm, w2_hbm, out_hbm,
                  idx1_v, idx2_v, w1_v, w2_v, rows1_v, rows2_v, out_v,
                  sem1, sem2):
    wid = lax.axis_index("s") * NC + lax.axis_index("c")
    base0 = wid * TPW
    for chunk in range(TPW // CH):
        base = base0 + chunk * CH
        pltpu.sync_copy(post_hbm.at[0, pl.ds(base, CH)], idx1_v)
        pltpu.sync_copy(post_hbm.at[1, pl.ds(base, CH)], idx2_v)
        pltpu.sync_copy(w1_hbm.at[pl.ds(base, CH)], w1_v)
        pltpu.sync_copy(w2_hbm.at[pl.ds(base, CH)], w2_v)
        cp1 = pltpu.async_copy(yw_hbm.at[idx1_v], rows1_v, sem1)
        cp2 = pltpu.async_copy(yw_hbm.at[idx2_v], rows2_v, sem2)
        cp1.wait()
        cp2.wait()

        def row_body(r, carry):
            w1s = w1_v[r, pl.ds(0, 16)]
            w2s = w2_v[r, pl.ds(0, 16)]
            for cc in range(D // 16):
                sl = pl.ds(cc * 16, 16)
                out_v[r, sl] = rows1_v[r, sl] * w1s + rows2_v[r, sl] * w2s
            return carry

        lax.fori_loop(0, CH, row_body, 0)
        pltpu.sync_copy(out_v, out_hbm.at[pl.ds(base, CH)])


def kernel(inputs, task_param, Wg_in, Wg_task, W1, W2, alpha):
    x = inputs.reshape(S, D)
    tp = task_param.reshape(S, D)
    a2 = jnp.reshape(alpha.astype(jnp.float32), (1, 1))

    post, w1c, w2c, te, laux = pl.pallas_call(
        _gate_body,
        in_specs=[
            pl.BlockSpec((S, D), lambda: (0, 0)),
            pl.BlockSpec((S, D), lambda: (0, 0)),
            pl.BlockSpec((D, E), lambda: (0, 0)),
            pl.BlockSpec((D, E), lambda: (0, 0)),
            pl.BlockSpec(memory_space=pltpu.SMEM),
        ],
        out_specs=[
            pl.BlockSpec((TOPK, S), lambda: (0, 0)),
            pl.BlockSpec((S, 16), lambda: (0, 0)),
            pl.BlockSpec((S, 16), lambda: (0, 0)),
            pl.BlockSpec((EP, 1), lambda: (0, 0)),
            pl.BlockSpec(memory_space=pltpu.SMEM),
        ],
        out_shape=[
            jax.ShapeDtypeStruct((TOPK, S), jnp.int32),
            jax.ShapeDtypeStruct((S, 16), jnp.float32),
            jax.ShapeDtypeStruct((S, 16), jnp.float32),
            jax.ShapeDtypeStruct((EP, 1), jnp.int32),
            jax.ShapeDtypeStruct((1, 1), jnp.float32),
        ],
    )(x, tp, Wg_in, Wg_task, a2)

    mesh = plsc.VectorSubcoreMesh(core_axis_name="c", subcore_axis_name="s")
    dispatch = functools.partial(
        pl.kernel, mesh=mesh,
        out_type=jax.ShapeDtypeStruct((NR, D), jnp.float32),
        scratch_types=[
            pltpu.VMEM((CHD,), jnp.int32),
            pltpu.VMEM((CHD,), jnp.int32),
            pltpu.VMEM((CHD, D), jnp.float32),
            pltpu.SemaphoreType.DMA,
            pltpu.SemaphoreType.DMA,
        ],
    )(_dispatch_body)
    xg = dispatch(x, post)

    yw = pl.pallas_call(
        _ffn_body,
        grid_spec=pltpu.PrefetchScalarGridSpec(
            num_scalar_prefetch=1,
            grid=(NT,),
            in_specs=[
                pl.BlockSpec((TILE, D), lambda t, te: (t, 0)),
                pl.BlockSpec((1, D, FF), lambda t, te: (te[t, 0], 0, 0)),
                pl.BlockSpec((1, FF, D), lambda t, te: (te[t, 0], 0, 0)),
            ],
            out_specs=pl.BlockSpec((TILE, D), lambda t, te: (t, 0)),
        ),
        out_shape=jax.ShapeDtypeStruct((NR, D), jnp.float32),
    )(te, xg, W1, W2)

    combine = functools.partial(
        pl.kernel, mesh=mesh,
        out_type=jax.ShapeDtypeStruct((S, D), jnp.float32),
        scratch_types=[
            pltpu.VMEM((CH,), jnp.int32),
            pltpu.VMEM((CH,), jnp.int32),
            pltpu.VMEM((CH, 16), jnp.float32),
            pltpu.VMEM((CH, 16), jnp.float32),
            pltpu.VMEM((CH, D), jnp.float32),
            pltpu.VMEM((CH, D), jnp.float32),
            pltpu.VMEM((CH, D), jnp.float32),
            pltpu.SemaphoreType.DMA,
            pltpu.SemaphoreType.DMA,
        ],
    )(_combine_body)
    out = combine(yw, post, w1c, w2c)

    return out.reshape(1, S, D), laux.reshape(())


# speedup vs baseline: 1.1159x; 1.0125x over previous
import functools

import jax
import jax.numpy as jnp
from jax import lax
from jax.experimental import pallas as pl
from jax.experimental.pallas import tpu as pltpu
from jax.experimental.pallas import tpu_sc as plsc

S, D, FF, E, TOPK = 2048, 1024, 2048, 8, 2
EP = 128
TILE = 128
NP = S * TOPK
NT = (NP + E * (TILE - 1) + TILE - 1) // TILE
NR = NT * TILE
_NEG = -1e30

NC, NS = 2, 16
NW = NC * NS
TPW = S // NW
CH = 32
CHD = 64


def _gate_body(x_ref, tp_ref, wgi_ref, wgt_ref, a_ref,
               post_ref, w1_ref, w2_ref, te_ref, laux_ref):
    a = a_ref[0, 0]
    gi = jnp.dot(x_ref[...], wgi_ref[...], preferred_element_type=jnp.float32)
    gt = jnp.dot(tp_ref[...], wgt_ref[...], preferred_element_type=jnp.float32)
    g = (1.0 - a) * gi + a * gt
    lane = lax.broadcasted_iota(jnp.int32, (S, E), 1)
    m1 = jnp.max(g, axis=1, keepdims=True)
    i1 = jnp.min(jnp.where(g == m1, lane, E), axis=1, keepdims=True)
    g2 = jnp.where(lane == i1, _NEG, g)
    m2 = jnp.max(g2, axis=1, keepdims=True)
    i2 = jnp.min(jnp.where(g2 == m2, lane, E), axis=1, keepdims=True)
    p = jnp.exp(g - m1)
    p = p / jnp.sum(p, axis=1, keepdims=True)
    aw = jnp.mean(p, axis=0, keepdims=True)
    cnt1 = (lane == i1).astype(jnp.float32)
    cnt2 = (lane == i2).astype(jnp.float32)
    ac = jnp.mean(cnt1 + cnt2, axis=0, keepdims=True)
    laux_ref[0, 0] = jnp.sum(aw * ac)
    e21 = jnp.exp(m2 - m1)
    w1_ref[...] = jnp.broadcast_to(1.0 / (1.0 + e21), (S, 16))
    w2_ref[...] = jnp.broadcast_to(e21 / (1.0 + e21), (S, 16))
    tot = (cnt1 + cnt2).astype(jnp.bfloat16)
    iota_r = lax.broadcasted_iota(jnp.int32, (S, S), 0)
    iota_c = lax.broadcasted_iota(jnp.int32, (S, S), 1)
    lower = (iota_c < iota_r).astype(jnp.bfloat16)
    P = jnp.dot(lower, tot, preferred_element_type=jnp.float32)
    counts = P[S - 1:S] + tot[S - 1:S].astype(jnp.float32)
    ptiles_f = jnp.ceil(counts * (1.0 / TILE))
    incl = (lax.broadcasted_iota(jnp.int32, (E, E), 0)
            <= lax.broadcasted_iota(jnp.int32, (E, E), 1)).astype(jnp.bfloat16)
    pe_t = jnp.dot(ptiles_f.astype(jnp.bfloat16), incl,
                   preferred_element_type=jnp.float32)
    pad_end = pe_t * TILE
    pad_off = pad_end - ptiles_f * TILE
    pos1 = jnp.sum(cnt1 * (pad_off + P), axis=1, keepdims=True)
    pos2 = jnp.sum(cnt2 * (pad_off + P), axis=1, keepdims=True)
    a1 = jnp.floor(pos1 * (1.0 / TILE))
    a2f = jnp.floor(pos2 * (1.0 / TILE))
    dig = jnp.concatenate([a1, pos1 - a1 * TILE, a2f, pos2 - a2f * TILE],
                          axis=1).astype(jnp.bfloat16)
    ident = (iota_c == iota_r).astype(jnp.bfloat16)
    digT = lax.dot_general(dig, ident, (((0,), (0,)), ((), ())),
                           preferred_element_type=jnp.float32)
    post_ref[...] = jnp.concatenate(
        [digT[0:1] * TILE + digT[1:2], digT[2:3] * TILE + digT[3:4]],
        axis=0).astype(jnp.int32)
    tvec = lax.broadcasted_iota(jnp.int32, (EP, E), 0).astype(jnp.float32) * TILE
    amat = (tvec >= pad_end).astype(jnp.int32)
    te = jnp.minimum(jnp.sum(amat, axis=1, keepdims=True), E - 1)
    nact = jnp.broadcast_to(pe_t[:, E - 1:E], (EP, 1)).astype(jnp.int32)
    rowi = lax.broadcasted_iota(jnp.int32, (EP, 1), 0)
    te_ref[...] = jnp.where(rowi == 64, nact, te)


def _dispatch_body(x_hbm, post_hbm, xg_hbm, idx1_v, idx2_v, rows_v, sem1, sem2):
    wid = lax.axis_index("s") * NC + lax.axis_index("c")
    base0 = wid * TPW
    for chunk in range(TPW // CHD):
        base = base0 + chunk * CHD
        pltpu.sync_copy(post_hbm.at[0, pl.ds(base, CHD)], idx1_v)
        pltpu.sync_copy(post_hbm.at[1, pl.ds(base, CHD)], idx2_v)
        pltpu.sync_copy(x_hbm.at[pl.ds(base, CHD)], rows_v)
        cp1 = pltpu.async_copy(rows_v, xg_hbm.at[idx1_v], sem1)
        cp2 = pltpu.async_copy(rows_v, xg_hbm.at[idx2_v], sem2)
        cp1.wait()
        cp2.wait()


def _ffn_body(te_ref, xg_ref, w1_ref, w2_ref, yw_ref):
    t = pl.program_id(0)

    @pl.when(t < te_ref[64, 0])
    def _():
        h = jnp.dot(xg_ref[...], w1_ref[0], preferred_element_type=jnp.float32)
        h = h * lax.logistic(h)
        yw_ref[...] = jnp.dot(h, w2_ref[0], preferred_element_type=jnp.float32)


def _combine_body(yw_hbm, post_hbm, w1_hbm, w2_hbm, out_hbm,
                  idx1_v, idx2_v, w1_v, w2_v, rows1_v, rows2_v, out_v,
                  sem1, sem2):
    wid = lax.axis_index("s") * NC + lax.axis_index("c")
    base0 = wid * TPW
    pltpu.sync_copy(post_hbm.at[0, pl.ds(base0, TPW)], idx1_v)
    pltpu.sync_copy(post_hbm.at[1, pl.ds(base0, TPW)], idx2_v)
    pltpu.sync_copy(w1_hbm.at[pl.ds(base0, TPW)], w1_v)
    pltpu.sync_copy(w2_hbm.at[pl.ds(base0, TPW)], w2_v)
    for chunk in range(TPW // CH):
        base = base0 + chunk * CH
        off = chunk * CH
        cp1 = pltpu.async_copy(yw_hbm.at[idx1_v.at[pl.ds(off, CH)]], rows1_v, sem1)
        cp2 = pltpu.async_copy(yw_hbm.at[idx2_v.at[pl.ds(off, CH)]], rows2_v, sem2)
        cp1.wait()
        cp2.wait()

        def row_body(r, carry):
            w1s = w1_v[off + r, pl.ds(0, 16)]
            w2s = w2_v[off + r, pl.ds(0, 16)]
            for cc in range(D // 16):
                sl = pl.ds(cc * 16, 16)
                out_v[r, sl] = rows1_v[r, sl] * w1s + rows2_v[r, sl] * w2s
            return carry

        lax.fori_loop(0, CH, row_body, 0)
        pltpu.sync_copy(out_v, out_hbm.at[pl.ds(base, CH)])


def kernel(inputs, task_param, Wg_in, Wg_task, W1, W2, alpha):
    x = inputs.reshape(S, D)
    tp = task_param.reshape(S, D)
    a2 = jnp.reshape(alpha.astype(jnp.float32), (1, 1))

    post, w1c, w2c, te, laux = pl.pallas_call(
        _gate_body,
        in_specs=[
            pl.BlockSpec((S, D), lambda: (0, 0)),
            pl.BlockSpec((S, D), lambda: (0, 0)),
            pl.BlockSpec((D, E), lambda: (0, 0)),
            pl.BlockSpec((D, E), lambda: (0, 0)),
            pl.BlockSpec(memory_space=pltpu.SMEM),
        ],
        out_specs=[
            pl.BlockSpec((TOPK, S), lambda: (0, 0)),
            pl.BlockSpec((S, 16), lambda: (0, 0)),
            pl.BlockSpec((S, 16), lambda: (0, 0)),
            pl.BlockSpec((EP, 1), lambda: (0, 0)),
            pl.BlockSpec(memory_space=pltpu.SMEM),
        ],
        out_shape=[
            jax.ShapeDtypeStruct((TOPK, S), jnp.int32),
            jax.ShapeDtypeStruct((S, 16), jnp.float32),
            jax.ShapeDtypeStruct((S, 16), jnp.float32),
            jax.ShapeDtypeStruct((EP, 1), jnp.int32),
            jax.ShapeDtypeStruct((1, 1), jnp.float32),
        ],
    )(x, tp, Wg_in, Wg_task, a2)

    mesh = plsc.VectorSubcoreMesh(core_axis_name="c", subcore_axis_name="s")
    dispatch = functools.partial(
        pl.kernel, mesh=mesh,
        out_type=jax.ShapeDtypeStruct((NR, D), jnp.float32),
        scratch_types=[
            pltpu.VMEM((CHD,), jnp.int32),
            pltpu.VMEM((CHD,), jnp.int32),
            pltpu.VMEM((CHD, D), jnp.float32),
            pltpu.SemaphoreType.DMA,
            pltpu.SemaphoreType.DMA,
        ],
    )(_dispatch_body)
    xg = dispatch(x, post)

    yw = pl.pallas_call(
        _ffn_body,
        grid_spec=pltpu.PrefetchScalarGridSpec(
            num_scalar_prefetch=1,
            grid=(NT,),
            in_specs=[
                pl.BlockSpec((TILE, D), lambda t, te: (t, 0)),
                pl.BlockSpec((1, D, FF), lambda t, te: (te[t, 0], 0, 0)),
                pl.BlockSpec((1, FF, D), lambda t, te: (te[t, 0], 0, 0)),
            ],
            out_specs=pl.BlockSpec((TILE, D), lambda t, te: (t, 0)),
        ),
        out_shape=jax.ShapeDtypeStruct((NR, D), jnp.float32),
    )(te, xg, W1, W2)

    combine = functools.partial(
        pl.kernel, mesh=mesh,
        out_type=jax.ShapeDtypeStruct((S, D), jnp.float32),
        scratch_types=[
            pltpu.VMEM((TPW,), jnp.int32),
            pltpu.VMEM((TPW,), jnp.int32),
            pltpu.VMEM((TPW, 16), jnp.float32),
            pltpu.VMEM((TPW, 16), jnp.float32),
            pltpu.VMEM((CH, D), jnp.float32),
            pltpu.VMEM((CH, D), jnp.float32),
            pltpu.VMEM((CH, D), jnp.float32),
            pltpu.SemaphoreType.DMA,
            pltpu.SemaphoreType.DMA,
        ],
    )(_combine_body)
    out = combine(yw, post, w1c, w2c)

    return out.reshape(1, S, D), laux.reshape(())
